# baseline (device time: 102178 ns/iter reference)
import functools

import jax
import jax.numpy as jnp
from jax import lax
from jax.experimental import pallas as pl
from jax.experimental.pallas import tpu as pltpu

N_DEV = 32
ROUNDS = 5
B = 16
H = 16
D = 64
BS = 16
NPAGES_LOCAL = 128
NB = 128
T = NPAGES_LOCAL * BS
NEG = -1e30


def kernel(Q, K, V, bt, lens):
    def body(q_ref, k_ref, v_ref, bt_ref, lens_ref, out_ref,
             cur_ref, recv_ref, send_sem, recv_sems):
        my = lax.axis_index("i")

        barrier_sem = pltpu.get_barrier_semaphore()
        for k in range(ROUNDS):
            partner = jnp.bitwise_xor(my, 1 << k)
            pl.semaphore_signal(
                barrier_sem, inc=1,
                device_id=(partner,), device_id_type=pl.DeviceIdType.MESH,
            )
        pl.semaphore_wait(barrier_sem, ROUNDS)

        lo = my * NPAGES_LOCAL
        bt_v = bt_ref[...]
        lp = bt_v - lo

        lens2d = jnp.concatenate(
            [jnp.full((1, 1), lens_ref[i], jnp.int32) for i in range(B)], axis=0
        )
        siota = lax.broadcasted_iota(jnp.int32, (B, NB), 1)
        valid = siota < lens2d

        piota = lax.broadcasted_iota(jnp.int32, (B, NPAGES_LOCAL, NB), 1)
        eq = (lp[:, None, :] == piota) & valid[:, None, :]
        cnt = jnp.sum(jnp.where(eq, 1.0, 0.0), axis=2)

        er = lax.broadcasted_iota(jnp.int32, (NPAGES_LOCAL, T), 0)
        ec = lax.broadcasted_iota(jnp.int32, (NPAGES_LOCAL, T), 1)
        E = jnp.where(ec // BS == er, 1.0, 0.0)
        cntk = lax.dot_general(
            cnt, E, (((1,), (0,)), ((), ())),
            preferred_element_type=jnp.float32,
        )
        attend = cntk > 0.0

        scale = D ** -0.5
        for h in range(H):
            q_h = q_ref[:, 0, h, :]
            k_h = k_ref[:, :, h, :].reshape(T, D)
            v_h = v_ref[:, :, h, :].reshape(T, D)
            s_h = lax.dot_general(
                q_h, k_h, (((1,), (1,)), ((), ())),
                preferred_element_type=jnp.float32,
            ) * scale
            s_h = jnp.where(attend, s_h, NEG)
            m_h = jnp.max(s_h, axis=1, keepdims=True)
            p_h = cntk * jnp.exp(s_h - m_h)
            l_h = jnp.sum(p_h, axis=1, keepdims=True)
            acc_h = lax.dot_general(
                p_h, v_h, (((1,), (0,)), ((), ())),
                preferred_element_type=jnp.float32,
            )
            cur_ref[0, :, h, :] = acc_h
            cur_ref[1, :, h, :] = jnp.broadcast_to(m_h, (B, D))
            cur_ref[2, :, h, :] = jnp.broadcast_to(l_h, (B, D))

        for k in range(ROUNDS):
            partner = jnp.bitwise_xor(my, 1 << k)
            rdma = pltpu.make_async_remote_copy(
                src_ref=cur_ref,
                dst_ref=recv_ref.at[k],
                send_sem=send_sem.at[0],
                recv_sem=recv_sems.at[k],
                device_id=(partner,),
                device_id_type=pl.DeviceIdType.MESH,
            )
            rdma.start()
            rdma.wait()

            a = cur_ref[...]
            b = recv_ref[k]
            m_new = jnp.maximum(a[1], b[1])
            sa = jnp.exp(a[1] - m_new)
            sb = jnp.exp(b[1] - m_new)
            cur_ref[0] = a[0] * sa + b[0] * sb
            cur_ref[1] = m_new
            cur_ref[2] = a[2] * sa + b[2] * sb

        out_ref[:, 0, :, :] = cur_ref[0] / cur_ref[2]

        @functools.partial(
            pl.run_scoped, second_barrier=pltpu.SemaphoreType.REGULAR
        )
        def _(second_barrier):
            for k in range(ROUNDS):
                partner = jnp.bitwise_xor(my, 1 << k)
                pl.semaphore_signal(
                    second_barrier, inc=1,
                    device_id=(partner,), device_id_type=pl.DeviceIdType.MESH,
                )
            pl.semaphore_wait(second_barrier, ROUNDS)

    return pl.pallas_call(
        body,
        out_shape=jax.ShapeDtypeStruct((B, 1, H, D), jnp.float32),
        in_specs=[
            pl.BlockSpec(memory_space=pltpu.VMEM),
            pl.BlockSpec(memory_space=pltpu.VMEM),
            pl.BlockSpec(memory_space=pltpu.VMEM),
            pl.BlockSpec(memory_space=pltpu.VMEM),
            pl.BlockSpec(memory_space=pltpu.SMEM),
        ],
        out_specs=pl.BlockSpec(memory_space=pltpu.VMEM),
        scratch_shapes=[
            pltpu.VMEM((3, B, H, D), jnp.float32),
            pltpu.VMEM((ROUNDS, 3, B, H, D), jnp.float32),
            pltpu.SemaphoreType.DMA((1,)),
            pltpu.SemaphoreType.DMA((ROUNDS,)),
        ],
        compiler_params=pltpu.CompilerParams(collective_id=0),
    )(Q, K, V, bt, lens)


# device time: 99928 ns/iter; 1.0225x vs baseline; 1.0225x over previous
import functools

import jax
import jax.numpy as jnp
from jax import lax
from jax.experimental import pallas as pl
from jax.experimental.pallas import tpu as pltpu

N_DEV = 32
ROUNDS = 5
B = 16
H = 16
D = 64
BS = 16
NPAGES_LOCAL = 128
NB = 128
T = NPAGES_LOCAL * BS
NEG = -1e30


def kernel(Q, K, V, bt, lens):
    def body(q_ref, k_ref, v_ref, bt_ref, lens_ref, out_ref,
             cur_ref, recv_ref, send_sem, recv_sems):
        my = lax.axis_index("i")

        barrier_sem = pltpu.get_barrier_semaphore()
        for k in range(ROUNDS):
            partner = jnp.bitwise_xor(my, 1 << k)
            pl.semaphore_signal(
                barrier_sem, inc=1,
                device_id=(partner,), device_id_type=pl.DeviceIdType.MESH,
            )
        pl.semaphore_wait(barrier_sem, ROUNDS)

        lo = my * NPAGES_LOCAL
        bt_v = bt_ref[...]
        lp = bt_v - lo

        lens2d = jnp.concatenate(
            [jnp.full((1, 1), lens_ref[i], jnp.int32) for i in range(B)], axis=0
        )
        siota = lax.broadcasted_iota(jnp.int32, (B, NB), 1)
        valid = siota < lens2d

        piota = lax.broadcasted_iota(jnp.int32, (B, NPAGES_LOCAL, NB), 1)
        eq = (lp[:, None, :] == piota) & valid[:, None, :]
        cnt = jnp.sum(jnp.where(eq, 1.0, 0.0), axis=2)

        er = lax.broadcasted_iota(jnp.int32, (NPAGES_LOCAL, T), 0)
        ec = lax.broadcasted_iota(jnp.int32, (NPAGES_LOCAL, T), 1)
        E = jnp.where(ec // BS == er, 1.0, 0.0)
        cntk = lax.dot_general(
            cnt, E, (((1,), (0,)), ((), ())),
            preferred_element_type=jnp.float32,
        )
        attend = cntk > 0.0

        scale = D ** -0.5
        for h in range(H):
            q_h = q_ref[:, 0, h, :].astype(jnp.bfloat16)
            k_h = k_ref[:, :, h, :].reshape(T, D).astype(jnp.bfloat16)
            v_h = v_ref[:, :, h, :].reshape(T, D).astype(jnp.bfloat16)
            s_h = lax.dot_general(
                q_h, k_h, (((1,), (1,)), ((), ())),
                preferred_element_type=jnp.float32,
            ) * scale
            s_h = jnp.where(attend, s_h, NEG)
            m_h = jnp.max(s_h, axis=1, keepdims=True)
            p_h = cntk * jnp.exp(s_h - m_h)
            l_h = jnp.sum(p_h, axis=1, keepdims=True)
            acc_h = lax.dot_general(
                p_h.astype(jnp.bfloat16), v_h, (((1,), (0,)), ((), ())),
                preferred_element_type=jnp.float32,
            )
            cur_ref[h, :, 0:D] = acc_h
            cur_ref[h, :, D:D + 1] = m_h
            cur_ref[h, :, D + 1:D + 2] = l_h

        for k in range(ROUNDS):
            partner = jnp.bitwise_xor(my, 1 << k)
            rdma = pltpu.make_async_remote_copy(
                src_ref=cur_ref,
                dst_ref=recv_ref.at[k],
                send_sem=send_sem.at[0],
                recv_sem=recv_sems.at[k],
                device_id=(partner,),
                device_id_type=pl.DeviceIdType.MESH,
            )
            rdma.start()
            rdma.wait()

            a = cur_ref[...]
            b = recv_ref[k]
            am = a[:, :, D:D + 1]
            bm = b[:, :, D:D + 1]
            m_new = jnp.maximum(am, bm)
            sa = jnp.exp(am - m_new)
            sb = jnp.exp(bm - m_new)
            cur_ref[:, :, 0:D] = a[:, :, 0:D] * sa + b[:, :, 0:D] * sb
            cur_ref[:, :, D:D + 1] = m_new
            cur_ref[:, :, D + 1:D + 2] = (
                a[:, :, D + 1:D + 2] * sa + b[:, :, D + 1:D + 2] * sb
            )

        for h in range(H):
            out_ref[:, 0, h, :] = (
                cur_ref[h, :, 0:D] / cur_ref[h, :, D + 1:D + 2]
            )

        @functools.partial(
            pl.run_scoped, second_barrier=pltpu.SemaphoreType.REGULAR
        )
        def _(second_barrier):
            for k in range(ROUNDS):
                partner = jnp.bitwise_xor(my, 1 << k)
                pl.semaphore_signal(
                    second_barrier, inc=1,
                    device_id=(partner,), device_id_type=pl.DeviceIdType.MESH,
                )
            pl.semaphore_wait(second_barrier, ROUNDS)

    return pl.pallas_call(
        body,
        out_shape=jax.ShapeDtypeStruct((B, 1, H, D), jnp.float32),
        in_specs=[
            pl.BlockSpec(memory_space=pltpu.VMEM),
            pl.BlockSpec(memory_space=pltpu.VMEM),
            pl.BlockSpec(memory_space=pltpu.VMEM),
            pl.BlockSpec(memory_space=pltpu.VMEM),
            pl.BlockSpec(memory_space=pltpu.SMEM),
        ],
        out_specs=pl.BlockSpec(memory_space=pltpu.VMEM),
        scratch_shapes=[
            pltpu.VMEM((H, B, 128), jnp.float32),
            pltpu.VMEM((ROUNDS, H, B, 128), jnp.float32),
            pltpu.SemaphoreType.DMA((1,)),
            pltpu.SemaphoreType.DMA((ROUNDS,)),
        ],
        compiler_params=pltpu.CompilerParams(collective_id=0),
    )(Q, K, V, bt, lens)
